# Initial kernel scaffold; baseline (speedup 1.0000x reference)
import jax
import jax.numpy as jnp
from jax.experimental import pallas as pl


def kernel(x, edge_index, edge_attr, batch, W1, b1, W2, b2, W3, b3):
    raise NotImplementedError("write your pallas kernel here")


# speedup vs baseline: 11.2952x; 11.2952x over previous
import functools

import jax
import jax.numpy as jnp
from jax import lax
from jax.experimental import pallas as pl
from jax.experimental.pallas import tpu as pltpu
from jax.experimental.pallas import tpu_sc as plsc

N = 10000
G = 64
H = 128
NC = 2
NS = 16
NW = NC * NS
CHUNK = 128
K = 80
EPAD = NW * K * CHUNK
NPAD = 10240
RPT = NPAD // NS
DW = 16

_mesh = plsc.VectorSubcoreMesh(core_axis_name="c", subcore_axis_name="s")

_dot = functools.partial(jnp.dot, precision=lax.Precision.HIGHEST,
                         preferred_element_type=jnp.float32)


@functools.partial(
    pl.kernel,
    out_type=jax.ShapeDtypeStruct((NC, NPAD, DW), jnp.float32),
    mesh=_mesh,
    scratch_types=[
        pltpu.VMEM((K, CHUNK), jnp.int32),
        pltpu.VMEM((CHUNK, DW), jnp.float32),
        pltpu.VMEM_SHARED((NPAD, DW), jnp.float32),
    ],
)
def _sc_degree(dstw_hbm, ones_hbm, zeros_hbm, out_hbm, idx_d, ones_v, acc):
    c = lax.axis_index("c")
    s = lax.axis_index("s")
    wid = s * NC + c
    pltpu.sync_copy(dstw_hbm.at[wid], idx_d)
    pltpu.sync_copy(ones_hbm, ones_v)
    pltpu.sync_copy(zeros_hbm.at[pl.ds(s * RPT, RPT)], acc.at[pl.ds(s * RPT, RPT)])
    plsc.subcore_barrier()

    def body(j, carry):
        pltpu.sync_copy(ones_v, acc.at[idx_d.at[j]], add=True)
        return carry

    lax.fori_loop(0, K, body, 0)
    plsc.subcore_barrier()
    pltpu.sync_copy(acc.at[pl.ds(s * RPT, RPT)], out_hbm.at[c, pl.ds(s * RPT, RPT)])


KH = K // 2


@functools.partial(
    pl.kernel,
    out_type=jax.ShapeDtypeStruct((NC, NPAD, H), jnp.float32),
    mesh=_mesh,
    scratch_types=[
        pltpu.VMEM((KH, CHUNK), jnp.int32),
        pltpu.VMEM((KH, CHUNK), jnp.int32),
        pltpu.VMEM((CHUNK, H), jnp.float32),
        pltpu.VMEM((CHUNK, H), jnp.float32),
        pltpu.VMEM_SHARED((NPAD, H), jnp.float32),
        pltpu.SemaphoreType.DMA,
        pltpu.SemaphoreType.DMA,
    ],
)
def _sc_aggregate(sx_hbm, srcw_hbm, dstw_hbm, zeros_hbm, out_hbm,
                  idx_s, idx_d, rows0, rows1, acc, sem0, sem1):
    c = lax.axis_index("c")
    s = lax.axis_index("s")
    wid = s * NC + c
    pltpu.sync_copy(zeros_hbm.at[pl.ds(s * RPT, RPT)], acc.at[pl.ds(s * RPT, RPT)])
    plsc.subcore_barrier()

    def body(i, carry):
        for b, (rows, sem) in enumerate(((rows0, sem0), (rows1, sem1))):
            j = i * 2 + b
            pltpu.make_async_copy(sx_hbm.at[idx_s.at[j]], rows, sem).wait()
            pltpu.sync_copy(rows, acc.at[idx_d.at[j]], add=True)
            jn = j + 2

            @pl.when(jn < KH)
            def _():
                pltpu.async_copy(sx_hbm.at[idx_s.at[jn]], rows, sem)
        return carry

    for half in range(K // KH):
        pltpu.sync_copy(srcw_hbm.at[wid, pl.ds(half * KH, KH)], idx_s)
        pltpu.sync_copy(dstw_hbm.at[wid, pl.ds(half * KH, KH)], idx_d)
        pltpu.async_copy(sx_hbm.at[idx_s.at[0]], rows0, sem0)
        pltpu.async_copy(sx_hbm.at[idx_s.at[1]], rows1, sem1)
        lax.fori_loop(0, KH // 2, body, 0)

    plsc.subcore_barrier()
    pltpu.sync_copy(acc.at[pl.ds(s * RPT, RPT)], out_hbm.at[c, pl.ds(s * RPT, RPT)])


def _tc_scale(deg0, deg1, x):
    def body(d0, d1, x_ref, sx_ref, dis_ref, deg_ref):
        deg = d0[:, 0:1] + d1[:, 0:1] + 1.0
        dis = lax.rsqrt(deg)
        deg_ref[...] = deg
        dis_ref[...] = dis
        sx_ref[...] = x_ref[...] * dis

    return pl.pallas_call(
        body,
        out_shape=[jax.ShapeDtypeStruct((N, H), jnp.float32),
                   jax.ShapeDtypeStruct((N, 1), jnp.float32),
                   jax.ShapeDtypeStruct((N, 1), jnp.float32)],
    )(deg0, deg1, x)


RB = 2000


def _row_spec(width):
    return pl.BlockSpec((RB, width), lambda i: (i, 0))


def _full_spec(r, c):
    return pl.BlockSpec((r, c), lambda i: (0, 0))


def _tc_layer1(a0, a1, x, dis, deg, W1, b1):
    def body(a0_ref, a1_ref, x_ref, dis_ref, deg_ref, w_ref, b_ref,
             y_ref, sy_ref):
        p = dis_ref[...] * (a0_ref[...] + a1_ref[...]) + x_ref[...] / deg_ref[...]
        z = _dot(p, w_ref[...]) + b_ref[...]
        y = jnp.maximum(z, 0.0)
        y_ref[...] = y
        sy_ref[...] = dis_ref[...] * y

    return pl.pallas_call(
        body,
        grid=(N // RB,),
        in_specs=[_row_spec(H), _row_spec(H), _row_spec(H), _row_spec(1),
                  _row_spec(1), _full_spec(H, H), _full_spec(1, H)],
        out_specs=[_row_spec(H), _row_spec(H)],
        out_shape=[jax.ShapeDtypeStruct((N, H), jnp.float32),
                   jax.ShapeDtypeStruct((N, H), jnp.float32)],
    )(a0, a1, x, dis, deg, W1, b1)


def _tc_layer2_pool(a0, a1, y, dis, deg, W2, b2, W3, b3, batch):
    def body(a0_ref, a1_ref, y_ref, dis_ref, deg_ref, w2_ref, b2_ref,
             w3_ref, b3_ref, batch_ref, out_ref, s_acc, c_acc):
        i = pl.program_id(0)
        p = dis_ref[...] * (a0_ref[...] + a1_ref[...]) + y_ref[...] / deg_ref[...]
        z = _dot(p, w2_ref[...]) + b2_ref[...]
        gids = lax.broadcasted_iota(jnp.int32, (G, RB), 0)
        onehot = jnp.where(batch_ref[0] == gids, 1.0, 0.0)
        ssum = _dot(onehot, z)
        cnt = jnp.sum(onehot, axis=1, keepdims=True)

        @pl.when(i == 0)
        def _():
            s_acc[...] = jnp.zeros_like(s_acc)
            c_acc[...] = jnp.zeros_like(c_acc)

        s_acc[...] += ssum
        c_acc[...] += cnt

        @pl.when(i == N // RB - 1)
        def _():
            pooled = s_acc[...] / jnp.maximum(c_acc[...], 1.0)
            out_ref[...] = _dot(pooled, w3_ref[...]) + b3_ref[...]

    return pl.pallas_call(
        body,
        grid=(N // RB,),
        in_specs=[_row_spec(H), _row_spec(H), _row_spec(H), _row_spec(1),
                  _row_spec(1), _full_spec(H, H), _full_spec(1, H),
                  _full_spec(H, 1), _full_spec(1, 1),
                  pl.BlockSpec((1, 1, RB), lambda i: (i, 0, 0))],
        out_specs=pl.BlockSpec((G, 1), lambda i: (0, 0)),
        out_shape=jax.ShapeDtypeStruct((G, 1), jnp.float32),
        scratch_shapes=[pltpu.VMEM((G, H), jnp.float32),
                        pltpu.VMEM((G, 1), jnp.float32)],
    )(a0, a1, y, dis, deg, W2, b2, W3, b3, batch)


def kernel(x, edge_index, edge_attr, batch, W1, b1, W2, b2, W3, b3):
    src = edge_index[0].astype(jnp.int32)
    dst = edge_index[1].astype(jnp.int32)
    e = src.shape[0]
    pad = EPAD - e
    srcw = jnp.concatenate([src, jnp.zeros((pad,), jnp.int32)]).reshape(NW, K, CHUNK)
    dstw = jnp.concatenate([dst, jnp.full((pad,), N, jnp.int32)]).reshape(NW, K, CHUNK)

    zeros_h = jnp.zeros((NPAD, H), jnp.float32)
    zeros_d = jnp.zeros((NPAD, DW), jnp.float32)
    ones_d = jnp.ones((CHUNK, DW), jnp.float32)

    degp = _sc_degree(dstw, ones_d, zeros_d)
    sx, dis, deg = _tc_scale(degp[0, :N], degp[1, :N], x)

    a1 = _sc_aggregate(sx, srcw, dstw, zeros_h)
    y, sy = _tc_layer1(a1[0, :N], a1[1, :N], x, dis, deg,
                       W1, b1.reshape(1, H))

    a2 = _sc_aggregate(sy, srcw, dstw, zeros_h)
    out = _tc_layer2_pool(a2[0, :N], a2[1, :N], y, dis, deg,
                          W2, b2.reshape(1, H), W3, b3.reshape(1, 1),
                          batch.astype(jnp.int32).reshape(N // RB, 1, RB))
    return out
